# speedup vs baseline: 1.6645x; 1.6645x over previous
import functools

import jax
import jax.numpy as jnp
from jax import lax
from jax.experimental import pallas as pl
from jax.experimental.pallas import tpu as pltpu
from jax.experimental.pallas import tpu_sc as plsc

_HIDDEN = 2048
_NUM_CORES = 2
_NUM_SUBCORES = 16
_NUM_WORKERS = _NUM_CORES * _NUM_SUBCORES
_CHUNK = 16
_NBUF = 3


def _emb_body(table_hbm, idx_hbm, out_hbm,
              idx_v, buf0, buf1, buf2, sem0, sem1, sem2):
    bufs = (buf0, buf1, buf2)
    sems = (sem0, sem1, sem2)
    wid = lax.axis_index("s") * _NUM_CORES + lax.axis_index("c")
    n_chunks = idx_hbm.shape[1]
    b_per_w = n_chunks * _CHUNK
    base = wid * b_per_w
    pltpu.sync_copy(idx_hbm.at[wid], idx_v)
    handles = {}
    for c in range(min(_NBUF, n_chunks)):
        handles[c] = pltpu.async_copy(
            table_hbm.at[idx_v.at[c]], bufs[c % _NBUF], sems[c % _NBUF])
    for c in range(n_chunks):
        p = c % _NBUF
        handles.pop(c).wait()
        pltpu.sync_copy(bufs[p], out_hbm.at[pl.ds(base + c * _CHUNK, _CHUNK)])
        nxt = c + _NBUF
        if nxt < n_chunks:
            handles[nxt] = pltpu.async_copy(
                table_hbm.at[idx_v.at[nxt]], bufs[p], sems[p])


def kernel(input_ids, embed_in_weight):
    out_shape = input_ids.shape + (_HIDDEN,)
    flat = input_ids.reshape(-1).astype(jnp.int32)
    total = flat.shape[0]
    b_per_w = total // _NUM_WORKERS
    n_chunks = b_per_w // _CHUNK
    idx3 = flat.reshape(_NUM_WORKERS, n_chunks, _CHUNK)
    mesh = plsc.VectorSubcoreMesh(core_axis_name="c", subcore_axis_name="s")
    run = functools.partial(
        pl.kernel,
        mesh=mesh,
        out_type=jax.ShapeDtypeStruct((total, _HIDDEN), jnp.float32),
        scratch_types=(
            [pltpu.VMEM((n_chunks, _CHUNK), jnp.int32)]
            + [pltpu.VMEM((_CHUNK, _HIDDEN), jnp.float32)] * _NBUF
            + [pltpu.SemaphoreType.DMA] * _NBUF
        ),
    )(_emb_body)
    out = run(embed_in_weight, idx3)
    return out.reshape(out_shape)
